# baseline (speedup 1.0000x reference)
import jax
import jax.numpy as jnp
from jax.experimental import pallas as pl

_B, _C, _N = 8, 190, 16384
_NB = 2048
_LANES = 128


def _conf_body(x_ref, out_ref):
    x = x_ref[...]
    m1 = jnp.max(x, axis=0)
    is_max = x == m1[None]
    cnt = jnp.sum(is_max.astype(jnp.float32), axis=0)
    neg = jnp.float32(-jnp.inf)
    m2c = jnp.max(jnp.where(is_max, neg, x), axis=0)
    m2 = jnp.where(cnt > 1.0, m1, m2c)
    conf = jnp.exp(1.0 - m1 / (m2 + 1e-8))
    psum = jnp.zeros((_B, _LANES), jnp.float32)
    for k in range(_NB // _LANES):
        psum = psum + conf[:, k * _LANES:(k + 1) * _LANES]
    out_ref[0] = psum


def kernel(sim_mat):
    xt = jnp.transpose(sim_mat, (1, 0, 2))
    nblk = _N // _NB
    psums = pl.pallas_call(
        _conf_body,
        grid=(nblk,),
        in_specs=[pl.BlockSpec((_C, _B, _NB), lambda n: (0, 0, n))],
        out_specs=pl.BlockSpec((1, _B, _LANES), lambda n: (n, 0, 0)),
        out_shape=jax.ShapeDtypeStruct((nblk, _B, _LANES), jnp.float32),
    )(xt)
    return psums.sum(axis=(0, 2)) / _N


# speedup vs baseline: 1.1943x; 1.1943x over previous
import jax
import jax.numpy as jnp
from jax.experimental import pallas as pl

_B, _C, _N = 8, 190, 16384
_NB = 2048


def _conf_body(x_ref, out_ref):
    m1 = x_ref[0]
    m2 = jnp.full((_B, _NB), -jnp.inf, jnp.float32)
    for c in range(1, _C):
        v = x_ref[c]
        m2 = jnp.maximum(m2, jnp.minimum(m1, v))
        m1 = jnp.maximum(m1, v)
    conf = jnp.exp(1.0 - m1 / (m2 + 1e-8))
    out_ref[0] = conf


def kernel(sim_mat):
    xt = jnp.transpose(sim_mat, (1, 0, 2))
    nblk = _N // _NB
    conf = pl.pallas_call(
        _conf_body,
        grid=(nblk,),
        in_specs=[pl.BlockSpec((_C, _B, _NB), lambda n: (0, 0, n))],
        out_specs=pl.BlockSpec((1, _B, _NB), lambda n: (n, 0, 0)),
        out_shape=jax.ShapeDtypeStruct((nblk, _B, _NB), jnp.float32),
    )(xt)
    return jnp.mean(conf, axis=(0, 2))
